# baseline (speedup 1.0000x reference)
import functools

import jax
import jax.numpy as jnp
from jax import lax
from jax.experimental import pallas as pl
from jax.experimental.pallas import tpu as pltpu
from jax.experimental.pallas import tpu_sc as plsc

NUM_HID = 64
BATCH = 1024
SEQ_LEN = 200

_NC = 2
_NS = 16
_NW = _NC * _NS
_SEQ_PER_W = BATCH // _NW
_CHUNK = 4
_NCHUNK = _SEQ_PER_W // _CHUNK
_ROWS = _CHUNK * SEQ_LEN

_SPB = 16
_PAIR = SEQ_LEN // 2


def _pos_encoding():
    positions = jnp.arange(SEQ_LEN, dtype=jnp.float32)[:, None]
    depth = NUM_HID / 2
    depths = jnp.arange(depth, dtype=jnp.float32)[None, :] / depth
    angle_rates = 1.0 / (10000.0 ** depths)
    angle_rads = positions * angle_rates
    return jnp.concatenate(
        [jnp.sin(angle_rads), jnp.cos(angle_rads)], axis=-1)


def _sc_body(x_hbm, tab_hbm, out_hbm, idx0, idx1, rows0, rows1,
             sem_g0, sem_g1, sem_s0, sem_s1):
    wid = lax.axis_index("s") * _NC + lax.axis_index("c")

    idxs = (idx0, idx1)
    rows = (rows0, rows1)
    sem_g = (sem_g0, sem_g1)
    sem_s = (sem_s0, sem_s1)
    gather_d = [None, None]
    scatter_d = [None, None]
    base_w = wid * _SEQ_PER_W * SEQ_LEN
    pair_w = wid * _SEQ_PER_W * _PAIR

    def scatter_chunk(g, b):
        d = None
        for s in range(_CHUNK):
            pb = pair_w + (g * _CHUNK + s) * _PAIR
            d = pltpu.async_copy(
                rows[b].at[pl.ds(s * SEQ_LEN, _PAIR)],
                out_hbm.at[pl.ds(pb, _PAIR), pl.ds(0, NUM_HID)], sem_s[b])
            d = pltpu.async_copy(
                rows[b].at[pl.ds(s * SEQ_LEN + _PAIR, _PAIR)],
                out_hbm.at[pl.ds(pb, _PAIR), pl.ds(NUM_HID, NUM_HID)],
                sem_s[b])
        return d

    def drain_chunk(b):
        for _ in range(2 * _CHUNK):
            scatter_d[b].wait()

    for g in range(_NCHUNK):
        b = g & 1
        base = base_w + g * _ROWS
        if scatter_d[b] is not None:
            drain_chunk(b)
        pltpu.sync_copy(x_hbm.at[pl.ds(base, _ROWS)], idxs[b])
        gather_d[b] = pltpu.async_copy(
            tab_hbm.at[idxs[b]], rows[b], sem_g[b])
        if g > 0:
            pb = 1 - b
            gather_d[pb].wait()
            scatter_d[pb] = scatter_chunk(g - 1, pb)

    last = (_NCHUNK - 1) & 1
    gather_d[last].wait()
    scatter_d[last] = scatter_chunk(_NCHUNK - 1, last)
    drain_chunk(1 - last)
    drain_chunk(last)


def _tc_body(g_ref, pe_ref, o_ref):
    x = g_ref[...]
    y = x.reshape(_SPB, _PAIR, 128) + pe_ref[...][None]
    o_ref[:, pl.Slice(0, _PAIR, 2), :] = y[:, :, :NUM_HID]
    o_ref[:, pl.Slice(1, _PAIR, 2), :] = y[:, :, NUM_HID:]


@jax.jit
def _run(x_perm, emb_table, pe_pair):
    mesh = plsc.VectorSubcoreMesh(
        core_axis_name="c", subcore_axis_name="s",
        num_cores=_NC, num_subcores=_NS)
    g2 = functools.partial(
        pl.kernel,
        out_type=jax.ShapeDtypeStruct((BATCH * _PAIR, 128), jnp.float32),
        mesh=mesh,
        scratch_types=[
            pltpu.VMEM((_ROWS,), jnp.int32),
            pltpu.VMEM((_ROWS,), jnp.int32),
            pltpu.VMEM((_ROWS, NUM_HID), jnp.float32),
            pltpu.VMEM((_ROWS, NUM_HID), jnp.float32),
            pltpu.SemaphoreType.DMA,
            pltpu.SemaphoreType.DMA,
            pltpu.SemaphoreType.DMA,
            pltpu.SemaphoreType.DMA,
        ],
        compiler_params=pltpu.CompilerParams(use_tc_tiling_on_sc=False),
    )(_sc_body)(x_perm, emb_table)

    return pl.pallas_call(
        _tc_body,
        grid=(BATCH // _SPB,),
        in_specs=[
            pl.BlockSpec((_SPB * _PAIR, 128), lambda i: (i, 0)),
            pl.BlockSpec((_PAIR, 128), lambda i: (0, 0)),
        ],
        out_specs=pl.BlockSpec((_SPB, SEQ_LEN, NUM_HID), lambda i: (i, 0, 0)),
        out_shape=jax.ShapeDtypeStruct((BATCH, SEQ_LEN, NUM_HID), jnp.float32),
    )(g2, pe_pair)


def kernel(x, emb_table):
    pe_pair = _pos_encoding().reshape(_PAIR, 128)
    x_perm = (x.astype(jnp.int32)
              .reshape(BATCH, _PAIR, 2)
              .transpose(0, 2, 1)
              .reshape(-1))
    return _run(x_perm, emb_table, pe_pair)


# speedup vs baseline: 1.4891x; 1.4891x over previous
import functools

import jax
import jax.numpy as jnp
from jax import lax
from jax.experimental import pallas as pl
from jax.experimental.pallas import tpu as pltpu
from jax.experimental.pallas import tpu_sc as plsc

NUM_HID = 64
BATCH = 1024
SEQ_LEN = 200

_NC = 2
_NS = 16
_NW = _NC * _NS
_SEQ_PER_W = BATCH // _NW
_CHUNK = 4
_NCHUNK = _SEQ_PER_W // _CHUNK
_ROWS = _CHUNK * SEQ_LEN

_PAIR = SEQ_LEN // 2
_PAIR_PAD = 104
_SEQ_PAD = 2 * _PAIR_PAD


def _pos_encoding():
    positions = jnp.arange(SEQ_LEN, dtype=jnp.float32)[:, None]
    depth = NUM_HID / 2
    depths = jnp.arange(depth, dtype=jnp.float32)[None, :] / depth
    angle_rates = 1.0 / (10000.0 ** depths)
    angle_rads = positions * angle_rates
    return jnp.concatenate(
        [jnp.sin(angle_rads), jnp.cos(angle_rads)], axis=-1)


def _sc_body(x_hbm, tab_hbm, out_hbm, idx0, idx1, rows0, rows1,
             sem_g0, sem_g1, sem_s0, sem_s1):
    wid = lax.axis_index("s") * _NC + lax.axis_index("c")

    idxs = (idx0, idx1)
    rows = (rows0, rows1)
    sem_g = (sem_g0, sem_g1)
    sem_s = (sem_s0, sem_s1)
    gather_d = [None, None]
    scatter_d = [None, None]
    base_w = wid * _SEQ_PER_W * SEQ_LEN
    pair_w = wid * _SEQ_PER_W * _PAIR

    def scatter_chunk(g, b):
        d = None
        for s in range(_CHUNK):
            seq = wid * _SEQ_PER_W + g * _CHUNK + s
            d = pltpu.async_copy(
                rows[b].at[pl.ds(s * SEQ_LEN, SEQ_LEN)],
                out_hbm.at[pl.ds(seq * _SEQ_PAD, SEQ_LEN)], sem_s[b])
        return d

    def drain_chunk(b):
        for _ in range(_CHUNK):
            scatter_d[b].wait()

    for g in range(_NCHUNK):
        b = g & 1
        base = base_w + g * _ROWS
        if scatter_d[b] is not None:
            drain_chunk(b)
        pltpu.sync_copy(x_hbm.at[pl.ds(base, _ROWS)], idxs[b])
        gather_d[b] = pltpu.async_copy(
            tab_hbm.at[idxs[b]], rows[b], sem_g[b])
        if g > 0:
            pb = 1 - b
            gather_d[pb].wait()
            scatter_d[pb] = scatter_chunk(g - 1, pb)

    last = (_NCHUNK - 1) & 1
    gather_d[last].wait()
    scatter_d[last] = scatter_chunk(_NCHUNK - 1, last)
    drain_chunk(1 - last)
    drain_chunk(last)


_BB = 128
_RB = _PAIR


def _tc_body(g_ref, pe_ref, o_ref):
    x = g_ref[:, :_PAIR, :]
    y = x + pe_ref[...][None]
    for r in range(_RB):
        for p in range(2):
            blk = y[:, r, p * NUM_HID:(p + 1) * NUM_HID]
            o_ref[2 * r + p] = blk.T


@jax.jit
def _run(x_perm, emb_table, pe_pair):
    mesh = plsc.VectorSubcoreMesh(
        core_axis_name="c", subcore_axis_name="s",
        num_cores=_NC, num_subcores=_NS)
    g2 = functools.partial(
        pl.kernel,
        out_type=jax.ShapeDtypeStruct((BATCH * _SEQ_PAD, NUM_HID),
                                      jnp.float32),
        mesh=mesh,
        scratch_types=[
            pltpu.VMEM((_ROWS,), jnp.int32),
            pltpu.VMEM((_ROWS,), jnp.int32),
            pltpu.VMEM((_ROWS, NUM_HID), jnp.float32),
            pltpu.VMEM((_ROWS, NUM_HID), jnp.float32),
            pltpu.SemaphoreType.DMA,
            pltpu.SemaphoreType.DMA,
            pltpu.SemaphoreType.DMA,
            pltpu.SemaphoreType.DMA,
        ],
        compiler_params=pltpu.CompilerParams(use_tc_tiling_on_sc=False),
    )(_sc_body)(x_perm, emb_table)

    g3 = g2.reshape(BATCH, _PAIR_PAD, 128)
    out_t = pl.pallas_call(
        _tc_body,
        grid=(BATCH // _BB,),
        in_specs=[
            pl.BlockSpec((_BB, _PAIR_PAD, 128), lambda i: (i, 0, 0)),
            pl.BlockSpec((_RB, 128), lambda i: (0, 0)),
        ],
        out_specs=pl.BlockSpec((SEQ_LEN, NUM_HID, _BB), lambda i: (0, 0, i)),
        out_shape=jax.ShapeDtypeStruct((SEQ_LEN, NUM_HID, BATCH), jnp.float32),
    )(g3, pe_pair)
    return out_t.transpose(2, 0, 1)


def kernel(x, emb_table):
    pe_pair = _pos_encoding().reshape(_PAIR, 128)
    x_flat = x.reshape(-1).astype(jnp.int32)
    return _run(x_flat, emb_table, pe_pair)


# speedup vs baseline: 1.6658x; 1.1186x over previous
import functools

import jax
import jax.numpy as jnp
from jax import lax
from jax.experimental import pallas as pl
from jax.experimental.pallas import tpu as pltpu
from jax.experimental.pallas import tpu_sc as plsc

NUM_HID = 64
BATCH = 1024
SEQ_LEN = 200

_NC = 2
_NS = 16
_NW = _NC * _NS
_SEQ_PER_W = BATCH // _NW
_CHUNK = 4
_NCHUNK = _SEQ_PER_W // _CHUNK
_ROWS = _CHUNK * SEQ_LEN

_PAIR = SEQ_LEN // 2
_PAIR_PAD = 104
_SEQ_PAD = 2 * _PAIR_PAD


def _pos_encoding():
    positions = jnp.arange(SEQ_LEN, dtype=jnp.float32)[:, None]
    depth = NUM_HID / 2
    depths = jnp.arange(depth, dtype=jnp.float32)[None, :] / depth
    angle_rates = 1.0 / (10000.0 ** depths)
    angle_rads = positions * angle_rates
    return jnp.concatenate(
        [jnp.sin(angle_rads), jnp.cos(angle_rads)], axis=-1)


def _sc_body(x_hbm, tab_hbm, out_hbm, idx0, idx1, rows0, rows1,
             sem_g0, sem_g1, sem_s0, sem_s1):
    wid = lax.axis_index("s") * _NC + lax.axis_index("c")

    idxs = (idx0, idx1)
    rows = (rows0, rows1)
    sem_g = (sem_g0, sem_g1)
    sem_s = (sem_s0, sem_s1)
    gather_d = [None, None]
    scatter_d = [None, None]
    base_w = wid * _SEQ_PER_W * SEQ_LEN
    pair_w = wid * _SEQ_PER_W * _PAIR

    def scatter_chunk(g, b):
        d = None
        for s in range(_CHUNK):
            seq = wid * _SEQ_PER_W + g * _CHUNK + s
            d = pltpu.async_copy(
                rows[b].at[pl.ds(s * SEQ_LEN, SEQ_LEN)],
                out_hbm.at[pl.ds(seq * _SEQ_PAD, SEQ_LEN)], sem_s[b])
        return d

    def drain_chunk(b):
        for _ in range(_CHUNK):
            scatter_d[b].wait()

    for g in range(_NCHUNK):
        b = g & 1
        base = base_w + g * _ROWS
        if scatter_d[b] is not None:
            drain_chunk(b)
        pltpu.sync_copy(x_hbm.at[pl.ds(base, _ROWS)], idxs[b])
        gather_d[b] = pltpu.async_copy(
            tab_hbm.at[idxs[b]], rows[b], sem_g[b])
        if g > 0:
            pb = 1 - b
            gather_d[pb].wait()
            scatter_d[pb] = scatter_chunk(g - 1, pb)

    last = (_NCHUNK - 1) & 1
    gather_d[last].wait()
    scatter_d[last] = scatter_chunk(_NCHUNK - 1, last)
    drain_chunk(1 - last)
    drain_chunk(last)


_BB = 128
_RB = _PAIR


def _tc_body(g_ref, pe_ref, o_ref):
    x = g_ref[:, :_PAIR, :]
    y = x + pe_ref[...][None]
    for r in range(_RB):
        t = y[:, r, :].T
        o_ref[2 * r] = t[:NUM_HID]
        o_ref[2 * r + 1] = t[NUM_HID:]


@jax.jit
def _run(x_perm, emb_table, pe_pair):
    mesh = plsc.VectorSubcoreMesh(
        core_axis_name="c", subcore_axis_name="s",
        num_cores=_NC, num_subcores=_NS)
    g2 = functools.partial(
        pl.kernel,
        out_type=jax.ShapeDtypeStruct((BATCH * _SEQ_PAD, NUM_HID),
                                      jnp.float32),
        mesh=mesh,
        scratch_types=[
            pltpu.VMEM((_ROWS,), jnp.int32),
            pltpu.VMEM((_ROWS,), jnp.int32),
            pltpu.VMEM((_ROWS, NUM_HID), jnp.float32),
            pltpu.VMEM((_ROWS, NUM_HID), jnp.float32),
            pltpu.SemaphoreType.DMA,
            pltpu.SemaphoreType.DMA,
            pltpu.SemaphoreType.DMA,
            pltpu.SemaphoreType.DMA,
        ],
        compiler_params=pltpu.CompilerParams(use_tc_tiling_on_sc=False),
    )(_sc_body)(x_perm, emb_table)

    g3 = g2.reshape(BATCH, _PAIR_PAD, 128)
    out_t = pl.pallas_call(
        _tc_body,
        grid=(BATCH // _BB,),
        in_specs=[
            pl.BlockSpec((_BB, _PAIR_PAD, 128), lambda i: (i, 0, 0)),
            pl.BlockSpec((_RB, 128), lambda i: (0, 0)),
        ],
        out_specs=pl.BlockSpec((SEQ_LEN, NUM_HID, _BB), lambda i: (0, 0, i)),
        out_shape=jax.ShapeDtypeStruct((SEQ_LEN, NUM_HID, BATCH), jnp.float32),
    )(g3, pe_pair)
    return out_t.transpose(2, 0, 1)


def kernel(x, emb_table):
    pe_pair = _pos_encoding().reshape(_PAIR, 128)
    x_flat = x.reshape(-1).astype(jnp.int32)
    return _run(x_flat, emb_table, pe_pair)


# speedup vs baseline: 1.7552x; 1.0537x over previous
import functools

import jax
import jax.numpy as jnp
from jax import lax
from jax.experimental import pallas as pl
from jax.experimental.pallas import tpu as pltpu
from jax.experimental.pallas import tpu_sc as plsc

NUM_HID = 64
NUM_VOCAB = 100000
BATCH = 1024
SEQ_LEN = 200

_NC = 2
_NS = 16
_NW = _NC * _NS
_SEQ_PER_W = BATCH // _NW
_CHUNK = 4
_NCHUNK = _SEQ_PER_W // _CHUNK
_ROWS = _CHUNK * SEQ_LEN

_PAIR = SEQ_LEN // 2
_PAIR_PAD = 104
_SEQ_PAD = 2 * _PAIR_PAD


def _pos_encoding():
    positions = jnp.arange(SEQ_LEN, dtype=jnp.float32)[:, None]
    depth = NUM_HID / 2
    depths = jnp.arange(depth, dtype=jnp.float32)[None, :] / depth
    angle_rates = 1.0 / (10000.0 ** depths)
    angle_rads = positions * angle_rates
    return jnp.concatenate(
        [jnp.sin(angle_rads), jnp.cos(angle_rads)], axis=-1)


def _sc_body(x_hbm, tab_hbm, out_hbm, idx0, idx1, rows0, rows1,
             sem_g0, sem_g1, sem_s0, sem_s1):
    wid = lax.axis_index("s") * _NC + lax.axis_index("c")

    idxs = (idx0, idx1)
    rows = (rows0, rows1)
    sem_g = (sem_g0, sem_g1)
    sem_s = (sem_s0, sem_s1)
    gather_d = [None, None]
    scatter_d = [None, None]
    base_w = wid * _SEQ_PER_W * SEQ_LEN
    pair_w = wid * _SEQ_PER_W * _PAIR

    def scatter_chunk(g, b):
        d = None
        for s in range(_CHUNK):
            seq = wid * _SEQ_PER_W + g * _CHUNK + s
            d = pltpu.async_copy(
                rows[b].at[pl.ds(s * SEQ_LEN, SEQ_LEN)],
                out_hbm.at[pl.ds(seq * _SEQ_PAD, SEQ_LEN)], sem_s[b])
        return d

    def drain_chunk(b):
        for _ in range(_CHUNK):
            scatter_d[b].wait()

    for g in range(_NCHUNK):
        b = g & 1
        base = base_w + g * _ROWS
        if scatter_d[b] is not None:
            drain_chunk(b)
        pltpu.sync_copy(x_hbm.at[pl.ds(base, _ROWS)], idxs[b])
        gather_d[b] = pltpu.async_copy(
            tab_hbm.at[idxs[b]], rows[b], sem_g[b])
        if g > 0:
            pb = 1 - b
            gather_d[pb].wait()
            scatter_d[pb] = scatter_chunk(g - 1, pb)

    last = (_NCHUNK - 1) & 1
    gather_d[last].wait()
    scatter_d[last] = scatter_chunk(_NCHUNK - 1, last)
    drain_chunk(1 - last)
    drain_chunk(last)


_BB = 128
_RB = _PAIR


def _tc_body(g_ref, pe_ref, o_ref):
    x = g_ref[:, :_PAIR, :]
    y = x + pe_ref[...][None]
    for r in range(_RB):
        t = y[:, r, :].T
        o_ref[2 * r] = t[:NUM_HID]
        o_ref[2 * r + 1] = t[NUM_HID:]


@jax.jit
def _run(x_perm, emb_table, pe_pair):
    mesh = plsc.VectorSubcoreMesh(
        core_axis_name="c", subcore_axis_name="s",
        num_cores=_NC, num_subcores=_NS)
    g2 = functools.partial(
        pl.kernel,
        out_type=jax.ShapeDtypeStruct((BATCH * _SEQ_PAD, NUM_HID),
                                      jnp.float32),
        mesh=mesh,
        scratch_types=[
            pltpu.VMEM((_ROWS,), jnp.int32),
            pltpu.VMEM((_ROWS,), jnp.int32),
            pltpu.VMEM((_ROWS, NUM_HID), jnp.float32),
            pltpu.VMEM((_ROWS, NUM_HID), jnp.float32),
            pltpu.SemaphoreType.DMA,
            pltpu.SemaphoreType.DMA,
            pltpu.SemaphoreType.DMA,
            pltpu.SemaphoreType.DMA,
        ],
        compiler_params=pltpu.CompilerParams(use_tc_tiling_on_sc=False),
    )(_sc_body)(x_perm, emb_table)

    g3 = g2.reshape(BATCH, _PAIR_PAD, 128)
    out_t = pl.pallas_call(
        _tc_body,
        grid=(BATCH // _BB,),
        in_specs=[
            pl.BlockSpec((_BB, _PAIR_PAD, 128), lambda i: (i, 0, 0)),
            pl.BlockSpec((_RB, 128), lambda i: (0, 0)),
        ],
        out_specs=pl.BlockSpec((SEQ_LEN, NUM_HID, _BB), lambda i: (0, 0, i)),
        out_shape=jax.ShapeDtypeStruct((SEQ_LEN, NUM_HID, BATCH), jnp.float32),
    )(g3, pe_pair)
    return out_t.transpose(2, 0, 1)


def kernel(x, emb_table):
    pe_pair = _pos_encoding().reshape(_PAIR, 128)
    tab2 = jnp.pad(emb_table, ((0, 0), (0, 64))).reshape(2 * NUM_VOCAB,
                                                         NUM_HID)
    x2 = x.reshape(-1).astype(jnp.int32) * 2
    return _run(x2, tab2, pe_pair)
